# baseline (speedup 1.0000x reference)
import math

import jax
import jax.numpy as jnp
from jax.experimental import pallas as pl
from jax.experimental.pallas import tpu as pltpu

_N = 10000
_NFEAT = 128
_NLAYERS = 8
_NHIDDEN = 64
_NCLASS = 16
_LAMDA = 0.5
_ALPHA = 0.1

_BM = 400
_NB = _N // _BM
_BMB = 400
_NBB = _N // _BMB
_HI = jax.lax.Precision.HIGHEST


def _split_cat(h):
    hi = h.astype(jnp.bfloat16)
    lo = (h - hi.astype(jnp.float32)).astype(jnp.bfloat16)
    return jnp.concatenate([hi, lo], axis=1)


def _body_a(x_ref, adj_ref, w0_ref, b0_ref, m_ref,
            adjbf_ref, h1c_ref, g0_ref, hc0, g0s):
    p = pl.program_id(0)
    m = pl.program_id(1)
    rows = pl.ds(m * _BM, _BM)

    @pl.when(p == 0)
    def _init():
        h0 = jax.nn.relu(
            jnp.dot(x_ref[...], w0_ref[...],
                    preferred_element_type=jnp.float32, precision=_HI)
            + b0_ref[...])
        g0s[rows, :] = _ALPHA * h0
        hc0[rows, :] = _split_cat(h0)

    @pl.when(p == 1)
    def _layer0():
        abf = adj_ref[...].astype(jnp.bfloat16)
        adjbf_ref[...] = abf
        r = jnp.dot(abf, hc0[...], preferred_element_type=jnp.float32)
        hi = r[:, :_NHIDDEN] + r[:, _NHIDDEN:]
        support = (1.0 - _ALPHA) * hi + g0s[rows, :]
        h1 = jax.nn.relu(
            jnp.dot(support, m_ref[0], preferred_element_type=jnp.float32,
                    precision=_HI))
        h1c_ref[...] = _split_cat(h1)
        g0_ref[...] = g0s[rows, :]


def _body_b(adjbf_ref, h1c_ref, g0_ref, m_ref, w1_ref, b1_ref,
            out_ref, hbuf):
    p = pl.program_id(0)
    m = pl.program_id(1)
    rows = pl.ds(m * _BMB, _BMB)

    def _step(rhs):
        r = jnp.dot(adjbf_ref[...], rhs, preferred_element_type=jnp.float32)
        hi = r[:, :_NHIDDEN] + r[:, _NHIDDEN:]
        support = (1.0 - _ALPHA) * hi + g0_ref[rows, :]
        return jax.nn.relu(
            jnp.dot(support, m_ref[p + 1], preferred_element_type=jnp.float32,
                    precision=_HI))

    @pl.when(p == 0)
    def _first():
        hbuf[0, rows, :] = _split_cat(_step(h1c_ref[...]))

    @pl.when(p > 0)
    def _rest():
        src = jax.lax.rem(p - 1, 2)
        hn = _step(hbuf[src])

        @pl.when(p < _NLAYERS - 2)
        def _store():
            hbuf[1 - src, rows, :] = _split_cat(hn)

        @pl.when(p == _NLAYERS - 2)
        def _final():
            out_ref[...] = jnp.dot(
                hn, w1_ref[...], preferred_element_type=jnp.float32,
                precision=_HI) + b1_ref[...]


def kernel(x, adj, adj_high, W_fc0, b_fc0, W_convs, W_fc1, b_fc1):
    del adj_high
    thetas = jnp.array(
        [math.log(_LAMDA / (i + 1) + 1.0) for i in range(_NLAYERS)],
        dtype=jnp.float32)
    eye = jnp.eye(_NHIDDEN, dtype=jnp.float32)
    M = thetas[:, None, None] * W_convs + (1.0 - thetas)[:, None, None] * eye

    adj_bf, h1c, g0 = pl.pallas_call(
        _body_a,
        grid=(2, _NB),
        in_specs=[
            pl.BlockSpec((_BM, _NFEAT), lambda p, m: (jnp.where(p == 0, m, 0), 0)),
            pl.BlockSpec((_BM, _N), lambda p, m: (jnp.where(p == 1, m, 0), 0)),
            pl.BlockSpec((_NFEAT, _NHIDDEN), lambda p, m: (0, 0)),
            pl.BlockSpec((1, _NHIDDEN), lambda p, m: (0, 0)),
            pl.BlockSpec((_NLAYERS, _NHIDDEN, _NHIDDEN), lambda p, m: (0, 0, 0)),
        ],
        out_specs=[
            pl.BlockSpec((_BM, _N), lambda p, m: (jnp.where(p == 1, m, 0), 0)),
            pl.BlockSpec((_BM, 2 * _NHIDDEN), lambda p, m: (jnp.where(p == 1, m, 0), 0)),
            pl.BlockSpec((_BM, _NHIDDEN), lambda p, m: (jnp.where(p == 1, m, 0), 0)),
        ],
        out_shape=[
            jax.ShapeDtypeStruct((_N, _N), jnp.bfloat16),
            jax.ShapeDtypeStruct((_N, 2 * _NHIDDEN), jnp.bfloat16),
            jax.ShapeDtypeStruct((_N, _NHIDDEN), jnp.float32),
        ],
        scratch_shapes=[
            pltpu.VMEM((_N, 2 * _NHIDDEN), jnp.bfloat16),
            pltpu.VMEM((_N, _NHIDDEN), jnp.float32),
        ],
        compiler_params=pltpu.CompilerParams(
            dimension_semantics=("arbitrary", "arbitrary")),
    )(x, adj, W_fc0, b_fc0[None, :], M)

    out = pl.pallas_call(
        _body_b,
        grid=(_NLAYERS - 1, _NBB),
        in_specs=[
            pl.BlockSpec((_BMB, _N), lambda p, m: (m, 0)),
            pl.BlockSpec((_N, 2 * _NHIDDEN), lambda p, m: (0, 0)),
            pl.BlockSpec((_N, _NHIDDEN), lambda p, m: (0, 0)),
            pl.BlockSpec((_NLAYERS, _NHIDDEN, _NHIDDEN), lambda p, m: (0, 0, 0)),
            pl.BlockSpec((_NHIDDEN, _NCLASS), lambda p, m: (0, 0)),
            pl.BlockSpec((1, _NCLASS), lambda p, m: (0, 0)),
        ],
        out_specs=pl.BlockSpec(
            (_BMB, _NCLASS), lambda p, m: (jnp.where(p == _NLAYERS - 2, m, 0), 0)),
        out_shape=jax.ShapeDtypeStruct((_N, _NCLASS), jnp.float32),
        scratch_shapes=[
            pltpu.VMEM((2, _N, 2 * _NHIDDEN), jnp.bfloat16),
        ],
        compiler_params=pltpu.CompilerParams(
            dimension_semantics=("arbitrary", "parallel")),
    )(adj_bf, h1c, g0, M, W_fc1, b_fc1[None, :])
    return out


# speedup vs baseline: 1.0179x; 1.0179x over previous
import math

import jax
import jax.numpy as jnp
from jax.experimental import pallas as pl
from jax.experimental.pallas import tpu as pltpu

_N = 10000
_NFEAT = 128
_NLAYERS = 8
_NHIDDEN = 64
_NCLASS = 16
_LAMDA = 0.5
_ALPHA = 0.1

_BM = 400
_NB = _N // _BM
_BMB = 400
_NBB = _N // _BMB
_HI = jax.lax.Precision.HIGHEST


def _split_cat(h):
    hi = h.astype(jnp.bfloat16)
    lo = (h - hi.astype(jnp.float32)).astype(jnp.bfloat16)
    return jnp.concatenate([hi, lo], axis=1)


def _body_a(x_ref, adj_ref, w0_ref, b0_ref, m_ref,
            adjbf_ref, h1c_ref, g0_ref, hc0, g0s):
    p = pl.program_id(0)
    m = pl.program_id(1)
    rows = pl.ds(m * _BM, _BM)

    @pl.when(p == 0)
    def _init():
        h0 = jax.nn.relu(
            jnp.dot(x_ref[...], w0_ref[...],
                    preferred_element_type=jnp.float32, precision=_HI)
            + b0_ref[...])
        g0s[rows, :] = _ALPHA * h0
        hc0[rows, :] = _split_cat(h0)

    @pl.when(p == 1)
    def _layer0():
        abf = adj_ref[...].astype(jnp.bfloat16)
        adjbf_ref[...] = abf
        r = jnp.dot(abf, hc0[...], preferred_element_type=jnp.float32)
        hi = r[:, :_NHIDDEN] + r[:, _NHIDDEN:]
        support = (1.0 - _ALPHA) * hi + g0s[rows, :]
        h1 = jax.nn.relu(
            jnp.dot(support, m_ref[0], preferred_element_type=jnp.float32,
                    precision=_HI))
        h1c_ref[...] = _split_cat(h1)
        g0_ref[...] = g0s[rows, :]


def _body_b(adjbf_ref, h1c_ref, g0_ref, m_ref, w1_ref, b1_ref,
            out_ref, hA, hB, rbuf):
    p = pl.program_id(0)
    m = pl.program_id(1)

    def _epi(r, mm, dst_ref):
        rows_mm = pl.ds(mm * _BMB, _BMB)
        hi = r[:, :_NHIDDEN] + r[:, _NHIDDEN:]
        support = (1.0 - _ALPHA) * hi + g0_ref[rows_mm, :]
        hn = jax.nn.relu(
            jnp.dot(support, m_ref[p + 1], preferred_element_type=jnp.float32,
                    precision=_HI))
        dst_ref[rows_mm, :] = _split_cat(hn)

    def _pipelined_phase(src_ref, dst_ref):
        @pl.when(m > 0)
        def _deferred():
            _epi(rbuf[...], m - 1, dst_ref)

        r = jnp.dot(adjbf_ref[...], src_ref[...],
                    preferred_element_type=jnp.float32)

        @pl.when(m < _NBB - 1)
        def _park():
            rbuf[...] = r

        @pl.when(m == _NBB - 1)
        def _flush():
            _epi(r, m, dst_ref)

    @pl.when(p == 0)
    def _first():
        _pipelined_phase(h1c_ref, hA)

    @pl.when((p > 0) & (p < _NLAYERS - 2) & (jax.lax.rem(p, 2) == 1))
    def _odd():
        _pipelined_phase(hA, hB)

    @pl.when((p > 0) & (p < _NLAYERS - 2) & (jax.lax.rem(p, 2) == 0))
    def _even():
        _pipelined_phase(hB, hA)

    @pl.when(p == _NLAYERS - 2)
    def _last():
        rows = pl.ds(m * _BMB, _BMB)
        r = jnp.dot(adjbf_ref[...], hB[...],
                    preferred_element_type=jnp.float32)
        hi = r[:, :_NHIDDEN] + r[:, _NHIDDEN:]
        support = (1.0 - _ALPHA) * hi + g0_ref[rows, :]
        hn = jax.nn.relu(
            jnp.dot(support, m_ref[p + 1], preferred_element_type=jnp.float32,
                    precision=_HI))
        out_ref[...] = jnp.dot(
            hn, w1_ref[...], preferred_element_type=jnp.float32,
            precision=_HI) + b1_ref[...]


def kernel(x, adj, adj_high, W_fc0, b_fc0, W_convs, W_fc1, b_fc1):
    del adj_high
    thetas = jnp.array(
        [math.log(_LAMDA / (i + 1) + 1.0) for i in range(_NLAYERS)],
        dtype=jnp.float32)
    eye = jnp.eye(_NHIDDEN, dtype=jnp.float32)
    M = thetas[:, None, None] * W_convs + (1.0 - thetas)[:, None, None] * eye

    adj_bf, h1c, g0 = pl.pallas_call(
        _body_a,
        grid=(2, _NB),
        in_specs=[
            pl.BlockSpec((_BM, _NFEAT), lambda p, m: (jnp.where(p == 0, m, 0), 0)),
            pl.BlockSpec((_BM, _N), lambda p, m: (jnp.where(p == 1, m, 0), 0)),
            pl.BlockSpec((_NFEAT, _NHIDDEN), lambda p, m: (0, 0)),
            pl.BlockSpec((1, _NHIDDEN), lambda p, m: (0, 0)),
            pl.BlockSpec((_NLAYERS, _NHIDDEN, _NHIDDEN), lambda p, m: (0, 0, 0)),
        ],
        out_specs=[
            pl.BlockSpec((_BM, _N), lambda p, m: (jnp.where(p == 1, m, 0), 0)),
            pl.BlockSpec((_BM, 2 * _NHIDDEN), lambda p, m: (jnp.where(p == 1, m, 0), 0)),
            pl.BlockSpec((_BM, _NHIDDEN), lambda p, m: (jnp.where(p == 1, m, 0), 0)),
        ],
        out_shape=[
            jax.ShapeDtypeStruct((_N, _N), jnp.bfloat16),
            jax.ShapeDtypeStruct((_N, 2 * _NHIDDEN), jnp.bfloat16),
            jax.ShapeDtypeStruct((_N, _NHIDDEN), jnp.float32),
        ],
        scratch_shapes=[
            pltpu.VMEM((_N, 2 * _NHIDDEN), jnp.bfloat16),
            pltpu.VMEM((_N, _NHIDDEN), jnp.float32),
        ],
        compiler_params=pltpu.CompilerParams(
            dimension_semantics=("arbitrary", "arbitrary")),
    )(x, adj, W_fc0, b_fc0[None, :], M)

    out = pl.pallas_call(
        _body_b,
        grid=(_NLAYERS - 1, _NBB),
        in_specs=[
            pl.BlockSpec((_BMB, _N), lambda p, m: (m, 0)),
            pl.BlockSpec((_N, 2 * _NHIDDEN), lambda p, m: (0, 0)),
            pl.BlockSpec((_N, _NHIDDEN), lambda p, m: (0, 0)),
            pl.BlockSpec((_NLAYERS, _NHIDDEN, _NHIDDEN), lambda p, m: (0, 0, 0)),
            pl.BlockSpec((_NHIDDEN, _NCLASS), lambda p, m: (0, 0)),
            pl.BlockSpec((1, _NCLASS), lambda p, m: (0, 0)),
        ],
        out_specs=pl.BlockSpec(
            (_BMB, _NCLASS), lambda p, m: (jnp.where(p == _NLAYERS - 2, m, 0), 0)),
        out_shape=jax.ShapeDtypeStruct((_N, _NCLASS), jnp.float32),
        scratch_shapes=[
            pltpu.VMEM((_N, 2 * _NHIDDEN), jnp.bfloat16),
            pltpu.VMEM((_N, 2 * _NHIDDEN), jnp.bfloat16),
            pltpu.VMEM((_BMB, 2 * _NHIDDEN), jnp.float32),
        ],
        compiler_params=pltpu.CompilerParams(
            dimension_semantics=("arbitrary", "arbitrary")),
    )(adj_bf, h1c, g0, M, W_fc1, b_fc1[None, :])
    return out
